# Initial kernel scaffold; baseline (speedup 1.0000x reference)
import jax
import jax.numpy as jnp
from jax.experimental import pallas as pl


def kernel(z, attention_mask, codebook):
    raise NotImplementedError("write your pallas kernel here")


# speedup vs baseline: 1.2201x; 1.2201x over previous
import functools

import jax
import jax.numpy as jnp
from jax import lax
from jax.experimental import pallas as pl
from jax.experimental.pallas import tpu as pltpu
from jax.experimental.pallas import tpu_sc as plsc

K = 8192
D = 256
N = 16 * 1024

BN = 512
SLAB_OFFS = (0, 2736, 5472)
SLAB_ENDS = (2736, 5472, 8192)


def _argmin_body(x_ref, x2_ref, w0_ref, w1_ref, w2_ref,
                 c0_ref, c1_ref, c2_ref, idx_ref):
    x = x_ref[...].astype(jnp.bfloat16)
    x2 = x2_ref[...]
    best = None
    bidx = None
    for w_ref, c_ref, off in zip((w0_ref, w1_ref, w2_ref),
                                 (c0_ref, c1_ref, c2_ref), SLAB_OFFS):
        mm = jax.lax.dot_general(
            x, w_ref[...].astype(jnp.bfloat16),
            (((1,), (0,)), ((), ())),
            preferred_element_type=jnp.float32,
        )
        s = (x2 - 2.0 * mm) + c_ref[...]
        m = jnp.min(s, axis=1, keepdims=True)
        cols = jax.lax.broadcasted_iota(jnp.int32, s.shape, 1) + off
        i = jnp.min(jnp.where(s == m, cols, jnp.int32(2**30)),
                    axis=1, keepdims=True)
        mr = m.astype(jnp.bfloat16).astype(jnp.float32)
        if best is None:
            best, bidx = mr, i
        else:
            take = m < best
            best = jnp.where(take, mr, best)
            bidx = jnp.where(take, i, bidx)
    idx_ref[...] = bidx


def _argmin_call(x, x2, wts, cns):
    nn = N // BN
    w_specs = [pl.BlockSpec((D, hi - lo), lambda n: (0, 0))
               for lo, hi in zip(SLAB_OFFS, SLAB_ENDS)]
    c_specs = [pl.BlockSpec((1, hi - lo), lambda n: (0, 0))
               for lo, hi in zip(SLAB_OFFS, SLAB_ENDS)]
    return pl.pallas_call(
        _argmin_body,
        grid=(nn,),
        in_specs=[
            pl.BlockSpec((BN, D), lambda n: (n, 0)),
            pl.BlockSpec((BN, 1), lambda n: (n, 0)),
            *w_specs,
            *c_specs,
        ],
        out_specs=pl.BlockSpec((BN, 1), lambda n: (n, 0)),
        out_shape=jax.ShapeDtypeStruct((N, 1), jnp.int32),
    )(x, x2, *wts, *cns)


def _make_sc_gather():
    info = plsc.get_sparse_core_info()
    nw = info.num_cores * info.num_subcores
    b_per_w = N // nw
    ch = 256
    nch = b_per_w // ch
    mesh = plsc.VectorSubcoreMesh(core_axis_name="c", subcore_axis_name="s")

    @functools.partial(
        pl.kernel,
        mesh=mesh,
        out_type=jax.ShapeDtypeStruct((N, D), jnp.float32),
        scratch_types=[
            pltpu.VMEM((ch,), jnp.int32),
            pltpu.VMEM((ch, D), jnp.float32),
            pltpu.SemaphoreType.DMA,
        ],
    )
    def gather(table_hbm, idx_hbm, out_hbm, idx_v, rows_v, sem):
        wid = lax.axis_index("s") * info.num_cores + lax.axis_index("c")
        base = wid * b_per_w
        for c in range(nch):
            off = base + c * ch
            pltpu.sync_copy(idx_hbm.at[pl.ds(off, ch)], idx_v)
            pltpu.async_copy(table_hbm.at[idx_v], rows_v, sem).wait()
            pltpu.sync_copy(rows_v, out_hbm.at[pl.ds(off, ch)])

    return gather


def kernel(z, attention_mask, codebook):
    del attention_mask
    x = z.reshape(N, D)
    wt = codebook.T
    x2 = jnp.sum(z**2, axis=-1).reshape(N, 1)
    cn = jnp.sum(codebook**2, axis=1).reshape(1, K)
    wts = [wt[:, lo:hi] for lo, hi in zip(SLAB_OFFS, SLAB_ENDS)]
    cns = [cn[:, lo:hi] for lo, hi in zip(SLAB_OFFS, SLAB_ENDS)]
    idx = _argmin_call(x, x2, wts, cns).reshape(N)
    quantized = _make_sc_gather()(codebook, idx)
    hidden_states = quantized.reshape(z.shape)
    return (hidden_states, idx)


# speedup vs baseline: 1.2449x; 1.0203x over previous
import functools

import jax
import jax.numpy as jnp
from jax import lax
from jax.experimental import pallas as pl
from jax.experimental.pallas import tpu as pltpu
from jax.experimental.pallas import tpu_sc as plsc

K = 8192
D = 256
N = 16 * 1024

BN = 512
SLAB_OFFS = (0, 2736, 5472)
SLAB_ENDS = (2736, 5472, 8192)


def _argmin_body(x_ref, x2_ref, w0_ref, w1_ref, w2_ref,
                 c0_ref, c1_ref, c2_ref, idx_ref):
    x = x_ref[...]
    x2 = x2_ref[...]
    best = None
    bidx = None
    for w_ref, c_ref, off in zip((w0_ref, w1_ref, w2_ref),
                                 (c0_ref, c1_ref, c2_ref), SLAB_OFFS):
        mm2 = jax.lax.dot_general(
            x, w_ref[...],
            (((1,), (0,)), ((), ())),
            preferred_element_type=jnp.float32,
        )
        s = (x2 + mm2) + c_ref[...]
        m = jnp.min(s, axis=1, keepdims=True)
        cols = jax.lax.broadcasted_iota(jnp.int32, s.shape, 1) + off
        i = jnp.min(jnp.where(s == m, cols, jnp.int32(2**30)),
                    axis=1, keepdims=True)
        mr = m.astype(jnp.bfloat16).astype(jnp.float32)
        if best is None:
            best, bidx = mr, i
        else:
            take = m < best
            best = jnp.where(take, mr, best)
            bidx = jnp.where(take, i, bidx)
    idx_ref[...] = bidx


def _argmin_call(x, x2, wts, cns):
    nn = N // BN
    w_specs = [pl.BlockSpec((D, hi - lo), lambda n: (0, 0))
               for lo, hi in zip(SLAB_OFFS, SLAB_ENDS)]
    c_specs = [pl.BlockSpec((1, hi - lo), lambda n: (0, 0))
               for lo, hi in zip(SLAB_OFFS, SLAB_ENDS)]
    return pl.pallas_call(
        _argmin_body,
        grid=(nn,),
        in_specs=[
            pl.BlockSpec((BN, D), lambda n: (n, 0)),
            pl.BlockSpec((BN, 1), lambda n: (n, 0)),
            *w_specs,
            *c_specs,
        ],
        out_specs=pl.BlockSpec((BN, 1), lambda n: (n, 0)),
        out_shape=jax.ShapeDtypeStruct((N, 1), jnp.int32),
    )(x, x2, *wts, *cns)


def _make_sc_gather():
    info = plsc.get_sparse_core_info()
    nw = info.num_cores * info.num_subcores
    b_per_w = N // nw
    ch = 256
    nch = b_per_w // ch
    mesh = plsc.VectorSubcoreMesh(core_axis_name="c", subcore_axis_name="s")

    @functools.partial(
        pl.kernel,
        mesh=mesh,
        out_type=jax.ShapeDtypeStruct((N, D), jnp.float32),
        scratch_types=[
            pltpu.VMEM((ch,), jnp.int32),
            pltpu.VMEM((ch, D), jnp.float32),
            pltpu.SemaphoreType.DMA,
        ],
    )
    def gather(table_hbm, idx_hbm, out_hbm, idx_v, rows_v, sem):
        wid = lax.axis_index("s") * info.num_cores + lax.axis_index("c")
        base = wid * b_per_w
        for c in range(nch):
            off = base + c * ch
            pltpu.sync_copy(idx_hbm.at[pl.ds(off, ch)], idx_v)
            pltpu.async_copy(table_hbm.at[idx_v], rows_v, sem).wait()
            pltpu.sync_copy(rows_v, out_hbm.at[pl.ds(off, ch)])

    return gather


def kernel(z, attention_mask, codebook):
    del attention_mask
    x = z.reshape(N, D)
    x_bf = x.astype(jnp.bfloat16)
    wt = (-2.0 * codebook.T).astype(jnp.bfloat16)
    x2 = jnp.sum(z**2, axis=-1).reshape(N, 1)
    cn = jnp.sum(codebook**2, axis=1).reshape(1, K)
    wts = [wt[:, lo:hi] for lo, hi in zip(SLAB_OFFS, SLAB_ENDS)]
    cns = [cn[:, lo:hi] for lo, hi in zip(SLAB_OFFS, SLAB_ENDS)]
    idx = _argmin_call(x_bf, x2, wts, cns).reshape(N)
    quantized = _make_sc_gather()(codebook, idx)
    hidden_states = quantized.reshape(z.shape)
    return (hidden_states, idx)


# speedup vs baseline: 1.3398x; 1.0762x over previous
import functools

import jax
import jax.numpy as jnp
from jax import lax
from jax.experimental import pallas as pl
from jax.experimental.pallas import tpu as pltpu
from jax.experimental.pallas import tpu_sc as plsc

K = 8192
D = 256
N = 16 * 1024

BN = 512
SLAB_OFFS = (0, 2736, 5472)
SLAB_ENDS = (2736, 5472, 8192)


def _argmin_body(x_ref, x2_ref, w0_ref, w1_ref, w2_ref,
                 c0_ref, c1_ref, c2_ref, cols_ref, idx_ref):
    x = x_ref[...]
    x2 = x2_ref[...]
    best = None
    bidx = None
    for w_ref, c_ref, off in zip((w0_ref, w1_ref, w2_ref),
                                 (c0_ref, c1_ref, c2_ref), SLAB_OFFS):
        mm2 = jax.lax.dot_general(
            x, w_ref[...],
            (((1,), (0,)), ((), ())),
            preferred_element_type=jnp.float32,
        )
        s = (x2 + mm2) + c_ref[...]
        m = jnp.min(s, axis=1, keepdims=True)
        cols = cols_ref[:, : s.shape[1]]
        i_f = jnp.min(jnp.where(s == m, cols, jnp.float32(3.0e38)),
                      axis=1, keepdims=True)
        i = i_f.astype(jnp.int32) + off
        mr = m.astype(jnp.bfloat16).astype(jnp.float32)
        if best is None:
            best, bidx = mr, i
        else:
            take = m < best
            best = jnp.where(take, mr, best)
            bidx = jnp.where(take, i, bidx)
    idx_ref[...] = bidx


def _argmin_call(x, x2, wts, cns, cols):
    nn = N // BN
    w_specs = [pl.BlockSpec((D, hi - lo), lambda n: (0, 0))
               for lo, hi in zip(SLAB_OFFS, SLAB_ENDS)]
    c_specs = [pl.BlockSpec((1, hi - lo), lambda n: (0, 0))
               for lo, hi in zip(SLAB_OFFS, SLAB_ENDS)]
    return pl.pallas_call(
        _argmin_body,
        grid=(nn,),
        in_specs=[
            pl.BlockSpec((BN, D), lambda n: (n, 0)),
            pl.BlockSpec((BN, 1), lambda n: (n, 0)),
            *w_specs,
            *c_specs,
            pl.BlockSpec((1, 2736), lambda n: (0, 0)),
        ],
        out_specs=pl.BlockSpec((BN, 1), lambda n: (n, 0)),
        out_shape=jax.ShapeDtypeStruct((N, 1), jnp.int32),
    )(x, x2, *wts, *cns, cols)


def _make_sc_gather():
    info = plsc.get_sparse_core_info()
    nw = info.num_cores * info.num_subcores
    b_per_w = N // nw
    ch = 256
    nch = b_per_w // ch
    mesh = plsc.VectorSubcoreMesh(core_axis_name="c", subcore_axis_name="s")

    @functools.partial(
        pl.kernel,
        mesh=mesh,
        out_type=jax.ShapeDtypeStruct((N, D), jnp.float32),
        scratch_types=[
            pltpu.VMEM((ch,), jnp.int32),
            pltpu.VMEM((ch, D), jnp.float32),
            pltpu.SemaphoreType.DMA,
        ],
    )
    def gather(table_hbm, idx_hbm, out_hbm, idx_v, rows_v, sem):
        wid = lax.axis_index("s") * info.num_cores + lax.axis_index("c")
        base = wid * b_per_w
        for c in range(nch):
            off = base + c * ch
            pltpu.sync_copy(idx_hbm.at[pl.ds(off, ch)], idx_v)
            pltpu.async_copy(table_hbm.at[idx_v], rows_v, sem).wait()
            pltpu.sync_copy(rows_v, out_hbm.at[pl.ds(off, ch)])

    return gather


def kernel(z, attention_mask, codebook):
    del attention_mask
    x = z.reshape(N, D)
    x_bf = x.astype(jnp.bfloat16)
    wt = (-2.0 * codebook.T).astype(jnp.bfloat16)
    x2 = jnp.sum(z**2, axis=-1).reshape(N, 1)
    cn = jnp.sum(codebook**2, axis=1).reshape(1, K)
    wts = [wt[:, lo:hi] for lo, hi in zip(SLAB_OFFS, SLAB_ENDS)]
    cns = [cn[:, lo:hi] for lo, hi in zip(SLAB_OFFS, SLAB_ENDS)]
    cols = jnp.arange(2736, dtype=jnp.float32).reshape(1, 2736)
    idx = _argmin_call(x_bf, x2, wts, cns, cols).reshape(N)
    quantized = _make_sc_gather()(codebook, idx)
    hidden_states = quantized.reshape(z.shape)
    return (hidden_states, idx)
